# baseline (speedup 1.0000x reference)
import functools

import jax
import jax.numpy as jnp
from jax import lax
from jax.experimental import pallas as pl
from jax.experimental.pallas import tpu as pltpu

TOPK = 64


def _extract_body(xt0, xt1, xt2, xt3, g0, g1, g2, g3, idx, dn_sc, *, n, k):
    x0 = xt0[...]
    x1 = xt1[...]
    dn_sc[...] = jnp.sqrt(x0 * x0 + x1 * x1 + 1e-6)
    br = x0.shape[1]
    jio = lax.broadcasted_iota(jnp.int32, (n, br), 0)
    big = jnp.float32(jnp.inf)

    def step(kk, carry):
        dn = dn_sc[...]
        m = jnp.min(dn, axis=0, keepdims=True)
        eq = dn == m
        jm = jnp.min(jnp.where(eq, jio, n), axis=0, keepdims=True)
        o = jio == jm
        dn_sc[...] = jnp.where(o, big, dn)
        idx[pl.ds(kk, 1), :] = jm
        zero = jnp.float32(0.0)
        g0[pl.ds(kk, 1), :] = jnp.sum(jnp.where(o, xt0[...], zero), axis=0, keepdims=True)
        g1[pl.ds(kk, 1), :] = jnp.sum(jnp.where(o, xt1[...], zero), axis=0, keepdims=True)
        g2[pl.ds(kk, 1), :] = jnp.sum(jnp.where(o, xt2[...], zero), axis=0, keepdims=True)
        g3[pl.ds(kk, 1), :] = jnp.sum(jnp.where(o, xt3[...], zero), axis=0, keepdims=True)
        return carry

    lax.fori_loop(0, k, step, 0)


def _extract_call(xt, *, n, k, br, interpret=False):
    nb = n // br
    body = functools.partial(_extract_body, n=n, k=k)
    in_spec = pl.BlockSpec((n, br), lambda i: (0, i))
    out_spec = pl.BlockSpec((k, br), lambda i: (0, i))
    f32 = jnp.float32
    return pl.pallas_call(
        body,
        grid=(nb,),
        in_specs=[in_spec] * 4,
        out_specs=[out_spec] * 5,
        out_shape=[
            jax.ShapeDtypeStruct((k, n), f32),
            jax.ShapeDtypeStruct((k, n), f32),
            jax.ShapeDtypeStruct((k, n), f32),
            jax.ShapeDtypeStruct((k, n), f32),
            jax.ShapeDtypeStruct((k, n), jnp.int32),
        ],
        scratch_shapes=[pltpu.VMEM((n, br), f32)],
        interpret=interpret,
    )(*xt)


def _mlp_body(fr, rr, w1r, b1r, w2r, b2r, w3r, b3r, w4tr, b4r, outr, maskr, *, s):
    f = fr[...]
    lio = lax.broadcasted_iota(jnp.int32, (s, 8), 1)
    a0 = f[:, 0:1]
    a1 = f[:, 1:2]
    d = jnp.sqrt(a0 * a0 + a1 * a1 + 1e-4)
    mask = (d <= 1.0).astype(jnp.float32)
    idxf = f[:, 4:5]
    rowf = f[:, 5:6]
    eye = (idxf == rowf).astype(jnp.float32)
    margin = d - rr[...]
    feat = jnp.where(lio == 4, eye, jnp.where(lio >= 5, margin, f))
    feat = jnp.where(lio >= 6, jnp.float32(0.0), feat)
    h = jnp.maximum(jnp.dot(feat, w1r[...], preferred_element_type=jnp.float32) + b1r[...], 0.0)
    h = jnp.maximum(jnp.dot(h, w2r[...], preferred_element_type=jnp.float32) + b2r[...], 0.0)
    h = jnp.maximum(jnp.dot(h, w3r[...], preferred_element_type=jnp.float32) + b3r[...], 0.0)
    o = jnp.sum(h * w4tr[...], axis=1, keepdims=True) + b4r[...]
    outr[...] = o * mask
    maskr[...] = mask


def _mlp_call(f8, r2, weights, *, n, k, s, interpret=False):
    nb = (n * k) // s
    body = functools.partial(_mlp_body, s=s)
    w1p, b1r, w2, b2r, w3, b3r, w4t, b4r = weights
    full = lambda a: pl.BlockSpec(a.shape, lambda b: tuple([0] * a.ndim))
    f32 = jnp.float32
    return pl.pallas_call(
        body,
        grid=(nb,),
        in_specs=[pl.BlockSpec((s, 8), lambda b: (b, 0)),
                  full(r2), full(w1p), full(b1r), full(w2),
                  full(b2r), full(w3), full(b3r), full(w4t), full(b4r)],
        out_specs=[pl.BlockSpec((s, 1), lambda b: (b, 0))] * 2,
        out_shape=[
            jax.ShapeDtypeStruct((n * k, 1), f32),
            jax.ShapeDtypeStruct((n * k, 1), f32),
        ],
        interpret=interpret,
    )(f8, r2, w1p, b1r, w2, b2r, w3, b3r, w4t, b4r)


def _run(x, r, W1, b1, W2, b2, W3, b3, W4, b4, *, k, br, s, interpret=False):
    n = x.shape[0]
    xt = [jnp.transpose(x[:, :, c]) for c in range(4)]
    g0, g1, g2, g3, idx_t = _extract_call(xt, n=n, k=k, br=br, interpret=interpret)
    indices = jnp.transpose(idx_t)
    rows = jnp.broadcast_to(jnp.arange(n, dtype=jnp.float32)[:, None], (n, k))
    f8 = jnp.stack(
        [jnp.transpose(g0), jnp.transpose(g1), jnp.transpose(g2), jnp.transpose(g3),
         indices.astype(jnp.float32), rows,
         jnp.zeros((n, k), jnp.float32), jnp.zeros((n, k), jnp.float32)],
        axis=-1).reshape(n * k, 8)
    w1p = jnp.zeros((8, W1.shape[1]), jnp.float32).at[:6, :].set(W1)
    weights = (w1p, b1.reshape(1, -1), W2, b2.reshape(1, -1), W3,
               b3.reshape(1, -1), W4.reshape(1, -1), b4.reshape(1, 1))
    out_f, mask_f = _mlp_call(f8, r.reshape(1, 1), weights, n=n, k=k, s=s,
                              interpret=interpret)
    return out_f.reshape(n, k, 1), mask_f.reshape(n, k, 1), indices


def kernel(x, r, W1, b1, W2, b2, W3, b3, W4, b4):
    return _run(x, r, W1, b1, W2, b2, W3, b3, W4, b4, k=TOPK, br=128, s=2048)


# speedup vs baseline: 2.1620x; 2.1620x over previous
import dataclasses
import functools

import jax
import jax.numpy as jnp
from jax import lax
from jax.experimental import pallas as pl
from jax.experimental.pallas import tpu as pltpu
from jax.experimental.pallas import tpu_sc as plsc

TOPK = 64


def _merge16(a_s, a_i, b_s, b_i):
    br_s = lax.rev(b_s, (0,))
    br_i = lax.rev(b_i, (0,))
    take = a_s <= br_s
    lo_s = jnp.where(take, a_s, br_s)
    lo_i = jnp.where(take, a_i, br_i)
    hi_s = jnp.where(take, br_s, a_s)
    hi_i = jnp.where(take, br_i, a_i)
    lo_s, lo_i = plsc.sort_key_val(lo_s, lo_i)
    hi_s, hi_i = plsc.sort_key_val(hi_s, hi_i)
    return lo_s, lo_i, hi_s, hi_i


def _sc_extract(x2d, *, n, k):
    info = plsc.get_sparse_core_info()
    nc, ns, L = info.num_cores, info.num_subcores, info.num_lanes
    nw = nc * ns
    rpw = n // nw
    nv = n // L
    nkv = k // L
    f32, i32 = jnp.float32, jnp.int32
    mesh = plsc.VectorSubcoreMesh(core_axis_name="core", subcore_axis_name="subcore")
    cp = pltpu.CompilerParams()
    if "needs_layout_passes" in pltpu.CompilerParams.__dataclass_fields__:
        cp = dataclasses.replace(cp, needs_layout_passes=False)

    @functools.partial(
        pl.kernel,
        compiler_params=cp,
        out_type=[
            jax.ShapeDtypeStruct((n, k), f32),
            jax.ShapeDtypeStruct((n, k), f32),
            jax.ShapeDtypeStruct((n, k), f32),
            jax.ShapeDtypeStruct((n, k), f32),
            jax.ShapeDtypeStruct((n, k), i32),
        ],
        mesh=mesh,
        scratch_types=[
            pltpu.VMEM((4 * n,), f32),
            pltpu.VMEM((4 * n,), f32),
            pltpu.VMEM((n,), f32),
            pltpu.VMEM((n + L,), f32),
            pltpu.VMEM((n + L,), i32),
            pltpu.VMEM((rpw, k), f32),
            pltpu.VMEM((rpw, k), f32),
            pltpu.VMEM((rpw, k), f32),
            pltpu.VMEM((rpw, k), f32),
            pltpu.VMEM((rpw, k), i32),
            pltpu.SemaphoreType.DMA,
            pltpu.SemaphoreType.DMA,
        ],
    )
    def ker(x_hbm, g0_hbm, g1_hbm, g2_hbm, g3_hbm, idx_hbm,
            buf_a, buf_b, s_buf, cand_s, cand_i,
            st0, st1, st2, st3, sti, sem_a, sem_b):
        wid = lax.axis_index("subcore") * nc + lax.axis_index("core")
        base = wid * rpw
        iota = lax.iota(i32, L)
        infv = jnp.full((L,), jnp.inf, f32)
        zerov = jnp.zeros((L,), i32)
        stages = (st0, st1, st2, st3)

        def process(rl, buf, sem):
            pltpu.make_async_copy(x_hbm.at[base + rl], buf, sem).wait()

            def p1(v, carry):
                m1, m2, m3, m4 = carry
                bidx = v * (4 * L) + 4 * iota
                x0 = plsc.load_gather(buf, [bidx])
                x1 = plsc.load_gather(buf, [bidx + 1])
                s = (x0 * x0 + x1 * x1) + jnp.float32(1e-6)
                s_buf[pl.ds(v * L, L)] = s
                m4n = jnp.minimum(m4, jnp.maximum(m3, s))
                m3n = jnp.minimum(m3, jnp.maximum(m2, s))
                m2n = jnp.minimum(m2, jnp.maximum(m1, s))
                m1n = jnp.minimum(m1, s)
                return (m1n, m2n, m3n, m4n)

            _, _, _, m4 = lax.fori_loop(0, nv, p1, (infv, infv, infv, infv))
            t = jnp.max(m4)

            def p2(v, cntv):
                sv = s_buf[pl.ds(v * L, L)]
                msk = sv <= t
                mi = jnp.where(msk, 1, 0)
                pos = cntv + plsc.cumsum(mi) - mi
                plsc.store_scatter(cand_s, [pos], sv, mask=msk)
                plsc.store_scatter(cand_i, [pos], iota + v * L, mask=msk)
                return cntv + plsc.all_reduce_population_count(msk)

            cntv = lax.fori_loop(0, nv, p2, zerov)
            cnt = jnp.max(cntv)
            nvec = (cnt + (L - 1)) // L

            def p3(ci, carry):
                bs0, bi0, bs1, bi1, bs2, bi2, bs3, bi3 = carry
                svec = cand_s[pl.ds(ci * L, L)]
                ivec = cand_i[pl.ds(ci * L, L)]
                valid = (iota + ci * L) < cntv
                svec = jnp.where(valid, svec, jnp.inf)
                zs, zi = plsc.sort_key_val(svec, ivec)
                bs0, bi0, zs, zi = _merge16(bs0, bi0, zs, zi)
                bs1, bi1, zs, zi = _merge16(bs1, bi1, zs, zi)
                bs2, bi2, zs, zi = _merge16(bs2, bi2, zs, zi)
                bs3, bi3, zs, zi = _merge16(bs3, bi3, zs, zi)
                return (bs0, bi0, bs1, bi1, bs2, bi2, bs3, bi3)

            init = (infv, zerov, infv, zerov, infv, zerov, infv, zerov)
            res = lax.fori_loop(0, nvec, p3, init)
            for j in range(nkv):
                bij = res[2 * j + 1]
                sti[rl, pl.ds(j * L, L)] = bij
                for c in range(4):
                    gc = plsc.load_gather(buf, [bij * 4 + c])
                    stages[c][rl, pl.ds(j * L, L)] = gc

        pltpu.make_async_copy(x_hbm.at[base], buf_a, sem_a).start()

        @pl.loop(0, rpw, step=2)
        def _(rl):
            pltpu.make_async_copy(x_hbm.at[base + rl + 1], buf_b, sem_b).start()
            process(rl, buf_a, sem_a)

            @pl.when(rl + 2 < rpw)
            def _():
                pltpu.make_async_copy(x_hbm.at[base + rl + 2], buf_a, sem_a).start()

            process(rl + 1, buf_b, sem_b)

        pltpu.sync_copy(st0, g0_hbm.at[pl.ds(base, rpw)])
        pltpu.sync_copy(st1, g1_hbm.at[pl.ds(base, rpw)])
        pltpu.sync_copy(st2, g2_hbm.at[pl.ds(base, rpw)])
        pltpu.sync_copy(st3, g3_hbm.at[pl.ds(base, rpw)])
        pltpu.sync_copy(sti, idx_hbm.at[pl.ds(base, rpw)])

    return ker(x2d)


def _mlp_body(fr, rr, w1r, b1r, w2r, b2r, w3r, b3r, w4tr, b4r, outr, maskr, *, s):
    f = fr[...]
    lio = lax.broadcasted_iota(jnp.int32, (s, 8), 1)
    a0 = f[:, 0:1]
    a1 = f[:, 1:2]
    d = jnp.sqrt(a0 * a0 + a1 * a1 + 1e-4)
    mask = (d <= 1.0).astype(jnp.float32)
    idxf = f[:, 4:5]
    rowf = f[:, 5:6]
    eye = (idxf == rowf).astype(jnp.float32)
    margin = d - rr[...]
    feat = jnp.where(lio == 4, eye, jnp.where(lio >= 5, margin, f))
    feat = jnp.where(lio >= 6, jnp.float32(0.0), feat)
    h = jnp.maximum(jnp.dot(feat, w1r[...], preferred_element_type=jnp.float32) + b1r[...], 0.0)
    h = jnp.maximum(jnp.dot(h, w2r[...], preferred_element_type=jnp.float32) + b2r[...], 0.0)
    h = jnp.maximum(jnp.dot(h, w3r[...], preferred_element_type=jnp.float32) + b3r[...], 0.0)
    o = jnp.sum(h * w4tr[...], axis=1, keepdims=True) + b4r[...]
    outr[...] = o * mask
    maskr[...] = mask


def _mlp_call(f8, r2, weights, *, n, k, s, interpret=False):
    nb = (n * k) // s
    body = functools.partial(_mlp_body, s=s)
    w1p, b1r, w2, b2r, w3, b3r, w4t, b4r = weights
    full = lambda a: pl.BlockSpec(a.shape, lambda b: tuple([0] * a.ndim))
    f32 = jnp.float32
    return pl.pallas_call(
        body,
        grid=(nb,),
        in_specs=[pl.BlockSpec((s, 8), lambda b: (b, 0)),
                  full(r2), full(w1p), full(b1r), full(w2),
                  full(b2r), full(w3), full(b3r), full(w4t), full(b4r)],
        out_specs=[pl.BlockSpec((s, 1), lambda b: (b, 0))] * 2,
        out_shape=[
            jax.ShapeDtypeStruct((n * k, 1), f32),
            jax.ShapeDtypeStruct((n * k, 1), f32),
        ],
        interpret=interpret,
    )(f8, r2, w1p, b1r, w2, b2r, w3, b3r, w4t, b4r)


def kernel(x, r, W1, b1, W2, b2, W3, b3, W4, b4):
    n = x.shape[0]
    k = TOPK
    x2d = x.reshape(n, 4 * n)
    g0, g1, g2, g3, indices = _sc_extract(x2d, n=n, k=k)
    rows = jnp.broadcast_to(jnp.arange(n, dtype=jnp.float32)[:, None], (n, k))
    zz = jnp.zeros((n, k), jnp.float32)
    f8 = jnp.stack([g0, g1, g2, g3, indices.astype(jnp.float32), rows, zz, zz],
                   axis=-1).reshape(n * k, 8)
    w1p = jnp.zeros((8, W1.shape[1]), jnp.float32).at[:6, :].set(W1)
    weights = (w1p, b1.reshape(1, -1), W2, b2.reshape(1, -1), W3,
               b3.reshape(1, -1), W4.reshape(1, -1), b4.reshape(1, 1))
    out_f, mask_f = _mlp_call(f8, r.reshape(1, 1), weights, n=n, k=k, s=2048)
    return out_f.reshape(n, k, 1), mask_f.reshape(n, k, 1), indices
